# Initial kernel scaffold; baseline (speedup 1.0000x reference)
import jax
import jax.numpy as jnp
from jax.experimental import pallas as pl


def kernel(x_user, x_item, edge_index_u2i, edge_index_i2u, params):
    raise NotImplementedError("write your pallas kernel here")


# speedup vs baseline: 1.0229x; 1.0229x over previous
import functools

import jax
import jax.numpy as jnp
from jax.experimental import pallas as pl

_HEADS = 4


def _mm_bias(x, W, b, bn=1000):
    N, K = x.shape
    F = W.shape[1]
    assert N % bn == 0, (N, bn)

    def body(x_ref, w_ref, b_ref, o_ref):
        o_ref[...] = (
            jnp.dot(x_ref[...], w_ref[...], preferred_element_type=jnp.float32)
            + b_ref[...]
        )

    return pl.pallas_call(
        body,
        grid=(N // bn,),
        in_specs=[
            pl.BlockSpec((bn, K), lambda i: (i, 0)),
            pl.BlockSpec((K, F), lambda i: (0, 0)),
            pl.BlockSpec((1, F), lambda i: (0, 0)),
        ],
        out_specs=pl.BlockSpec((bn, F), lambda i: (i, 0)),
        out_shape=jax.ShapeDtypeStruct((N, F), jnp.float32),
    )(x, W, b.reshape(1, F))


def _gatv2_xla(xl, xr, edge_index, att, bias, num_dst):
    heads = _HEADS
    hf = xl.shape[1]
    fh = hf // heads
    xl_h = xl.reshape(-1, heads, fh)
    xr_h = xr.reshape(-1, heads, fh)
    src = edge_index[0]
    dst = edge_index[1]
    m = jax.nn.leaky_relu(xl_h[src] + xr_h[dst], negative_slope=0.2)
    e = jnp.sum(m * att[None, :, :], axis=-1)
    a = jnp.exp(e)
    den = jax.ops.segment_sum(a, dst, num_segments=num_dst)
    out = jax.ops.segment_sum(xl_h[src] * a[:, :, None], dst, num_segments=num_dst)
    out = out / (den[:, :, None] + 1e-16)
    return out.reshape(num_dst, hf) + bias


def _layer_norm(x, g, b):
    mu = jnp.mean(x, axis=-1, keepdims=True)
    var = jnp.var(x, axis=-1, keepdims=True)
    return (x - mu) / jnp.sqrt(var + 1e-5) * g + b


def kernel(x_user, x_item, edge_index_u2i, edge_index_i2u, params):
    NU = x_user.shape[0]
    NI = x_item.shape[0]
    h = {
        'user': _mm_bias(x_user, params['proj']['user']['W'], params['proj']['user']['b']),
        'item': _mm_bias(x_item, params['proj']['item']['W'], params['proj']['item']['b']),
    }
    n_layers = len(params['layers'])
    for i, layer in enumerate(params['layers']):
        pu = layer['convs']['u2i']
        pi = layer['convs']['i2u']
        xl_u = _mm_bias(h['user'], pu['lin_l']['W'], pu['lin_l']['b'])
        xr_i = _mm_bias(h['item'], pu['lin_r']['W'], pu['lin_r']['b'])
        xl_i = _mm_bias(h['item'], pi['lin_l']['W'], pi['lin_l']['b'])
        xr_u = _mm_bias(h['user'], pi['lin_r']['W'], pi['lin_r']['b'])
        out_item = _gatv2_xla(xl_u, xr_i, edge_index_u2i, pu['att'], pu['bias'], NI)
        out_user = _gatv2_xla(xl_i, xr_u, edge_index_i2u, pi['att'], pi['bias'], NU)
        h_new = {'user': out_user, 'item': out_item}
        for nt in ('user', 'item'):
            n = layer['norms'][nt]
            z = _layer_norm(h_new[nt], n['g'], n['b'])
            if h[nt].shape == z.shape:
                z = z + h[nt]
            if i < n_layers - 1:
                z = jax.nn.elu(z)
            h_new[nt] = z
        h = h_new
    return h['user'], h['item']
